# Initial kernel scaffold; baseline (speedup 1.0000x reference)
import jax
import jax.numpy as jnp
from jax import lax
from jax.experimental import pallas as pl
from jax.experimental.pallas import tpu as pltpu
from jax.experimental.pallas import tpu_sc as plsc

H = 128
DW = H + 16
CHUNK = 128
NC = 2
NS = 16
BLK = 1280


def _pre_body(h_ref, at_ref, bt_ref, b1_ref, p_ref, q_ref):
    x = h_ref[...]
    p_ref[...] = jnp.dot(x, at_ref[...], preferred_element_type=jnp.float32)
    q_ref[...] = (jnp.dot(x, bt_ref[...], preferred_element_type=jnp.float32)
                  + b1_ref[...])


def _post_body(h_ref, s0_ref, s1_ref, w2e_ref, wua_ref, wub_ref, b1u_ref,
               wu2_ref, b2u_ref, o_ref):
    s = s0_ref[...] + s1_ref[...]
    agg = jnp.dot(s, w2e_ref[...], preferred_element_type=jnp.float32)
    z = jnp.dot(h_ref[...], wua_ref[...], preferred_element_type=jnp.float32)
    z = z + jnp.dot(agg, wub_ref[...], preferred_element_type=jnp.float32)
    z = jnp.maximum(z + b1u_ref[...], 0.0)
    o_ref[...] = (h_ref[...]
                  + jnp.dot(z, wu2_ref[...], preferred_element_type=jnp.float32)
                  + b2u_ref[...])


def _make_edge_kernel(np_rows, chunks_per_worker):
    rows_per_tile = np_rows // NS
    mesh = plsc.VectorSubcoreMesh(core_axis_name="c", subcore_axis_name="s")

    def body(p_hbm, q_hbm, src_hbm, dst_hbm, out_hbm,
             src_v, dst_v, prow, qrow, orow, s_sh, sem_p, sem_q):
        cid = lax.axis_index("c")
        sid = lax.axis_index("s")
        w = sid * NC + cid

        zero16 = jnp.zeros((16,), jnp.float32)

        def zero_row(i, carry):
            for j in range(DW // 16):
                orow[i, pl.ds(j * 16, 16)] = zero16
            return carry

        lax.fori_loop(0, CHUNK, zero_row, 0)

        for t in range(rows_per_tile // CHUNK):
            pltpu.sync_copy(
                orow, s_sh.at[pl.ds(sid * rows_per_tile + t * CHUNK, CHUNK)])
        plsc.subcore_barrier()

        lane = lax.iota(jnp.int32, 16)
        unit = jnp.where(lane == 0, 1.0, 0.0).astype(jnp.float32)

        def const_row(i, carry):
            orow[i, pl.ds(H, 16)] = unit
            return carry

        lax.fori_loop(0, CHUNK, const_row, 0)

        def chunk_body(k, carry):
            base = (w * chunks_per_worker + k) * CHUNK
            pltpu.sync_copy(src_hbm.at[pl.ds(base, CHUNK)], src_v)
            pltpu.sync_copy(dst_hbm.at[pl.ds(base, CHUNK)], dst_v)
            cp = pltpu.async_copy(p_hbm.at[src_v], prow, sem_p)
            cq = pltpu.async_copy(q_hbm.at[dst_v], qrow, sem_q)
            cp.wait()
            cq.wait()

            def relu_row(i, c2):
                for j in range(H // 16):
                    s = pl.ds(j * 16, 16)
                    orow[i, s] = jnp.maximum(prow[i, s] + qrow[i, s], 0.0)
                return c2

            lax.fori_loop(0, CHUNK, relu_row, 0)
            pltpu.sync_copy(orow, s_sh.at[dst_v], add=True)
            return carry

        lax.fori_loop(0, chunks_per_worker, chunk_body, 0)
        plsc.subcore_barrier()

        for t in range(rows_per_tile // CHUNK):
            r0 = sid * rows_per_tile + t * CHUNK
            pltpu.sync_copy(s_sh.at[pl.ds(r0, CHUNK)], orow)
            pltpu.sync_copy(orow, out_hbm.at[cid, pl.ds(r0, CHUNK)])

    return pl.kernel(
        body,
        mesh=mesh,
        out_type=jax.ShapeDtypeStruct((NC, np_rows, DW), jnp.float32),
        scratch_types=[
            pltpu.VMEM((CHUNK,), jnp.int32),
            pltpu.VMEM((CHUNK,), jnp.int32),
            pltpu.VMEM((CHUNK, H), jnp.float32),
            pltpu.VMEM((CHUNK, H), jnp.float32),
            pltpu.VMEM((CHUNK, DW), jnp.float32),
            pltpu.VMEM_SHARED((np_rows, DW), jnp.float32),
            pltpu.SemaphoreType.DMA,
            pltpu.SemaphoreType.DMA,
        ],
    )


def kernel(h, edge_index, W1m, b1m, W2m, b2m, W1u, b1u, W2u, b2u):
    n = h.shape[1]
    e = edge_index.shape[1]
    np_rows = -(-(n + 1) // (NS * CHUNK)) * (NS * CHUNK)
    edges_per_wave = NC * NS * CHUNK
    e_pad = -(-e // edges_per_wave) * edges_per_wave
    chunks_per_worker = e_pad // edges_per_wave

    h0 = h[0]
    h_pad = jnp.concatenate(
        [h0, jnp.zeros((np_rows - n, H), jnp.float32)], axis=0)
    src = jnp.concatenate(
        [edge_index[0].astype(jnp.int32),
         jnp.full((e_pad - e,), n, jnp.int32)])
    dst = jnp.concatenate(
        [edge_index[1].astype(jnp.int32),
         jnp.full((e_pad - e,), n, jnp.int32)])

    at = W1m[:, :H].T
    bt = W1m[:, H:].T

    grid = (np_rows // BLK,)
    full = lambda i: (0, 0)
    rowblk = pl.BlockSpec((BLK, H), lambda i: (i, 0))
    p_nodes, q_nodes = pl.pallas_call(
        _pre_body,
        grid=grid,
        in_specs=[rowblk,
                  pl.BlockSpec((H, H), full),
                  pl.BlockSpec((H, H), full),
                  pl.BlockSpec((1, H), full)],
        out_specs=[rowblk, rowblk],
        out_shape=[jax.ShapeDtypeStruct((np_rows, H), jnp.float32)] * 2,
    )(h_pad, at, bt, b1m.reshape(1, H))

    s_part = _make_edge_kernel(np_rows, chunks_per_worker)(
        p_nodes, q_nodes, src, dst)

    w2ext = jnp.concatenate(
        [W2m.T, b2m.reshape(1, H), jnp.zeros((DW - H - 1, H), jnp.float32)],
        axis=0)
    sblk = pl.BlockSpec((BLK, DW), lambda i: (i, 0))
    out_full = pl.pallas_call(
        _post_body,
        grid=grid,
        in_specs=[rowblk, sblk, sblk,
                  pl.BlockSpec((DW, H), full),
                  pl.BlockSpec((H, H), full),
                  pl.BlockSpec((H, H), full),
                  pl.BlockSpec((1, H), full),
                  pl.BlockSpec((H, H), full),
                  pl.BlockSpec((1, H), full)],
        out_specs=rowblk,
        out_shape=jax.ShapeDtypeStruct((np_rows, H), jnp.float32),
    )(h_pad, s_part[0], s_part[1], w2ext, W1u[:, :H].T, W1u[:, H:].T,
      b1u.reshape(1, H), W2u.T, b2u.reshape(1, H))

    return out_full[:n][None]


# speedup vs baseline: 2.5483x; 2.5483x over previous
import jax
import jax.numpy as jnp
from jax import lax
from jax.experimental import pallas as pl
from jax.experimental.pallas import tpu as pltpu
from jax.experimental.pallas import tpu_sc as plsc

H = 128
HW = 64
DW = 80
CHUNK = 128
NC = 2
NS = 16
BLK = 1280


def _pre_body(h_ref, at_ref, bt_ref, b1_ref, p_ref, q_ref):
    x = h_ref[...]
    rp = jnp.dot(x, at_ref[...], preferred_element_type=jnp.float32)
    rq = (jnp.dot(x, bt_ref[...], preferred_element_type=jnp.float32)
          + b1_ref[...])
    p_ref[0] = rp[:, :HW]
    p_ref[1] = rp[:, HW:]
    q_ref[0] = rq[:, :HW]
    q_ref[1] = rq[:, HW:]


def _post_body(h_ref, s0_ref, s1_ref, w2a_ref, w2b_ref, b2m_ref, wua_ref,
               wub_ref, b1u_ref, wu2_ref, b2u_ref, o_ref):
    s0 = s0_ref[...]
    s1 = s1_ref[...]
    agg = jnp.dot(s0[:, :HW], w2a_ref[...], preferred_element_type=jnp.float32)
    agg = agg + jnp.dot(s1[:, :HW], w2b_ref[...],
                        preferred_element_type=jnp.float32)
    agg = agg + s0[:, HW:HW + 1] * b2m_ref[...]
    z = jnp.dot(h_ref[...], wua_ref[...], preferred_element_type=jnp.float32)
    z = z + jnp.dot(agg, wub_ref[...], preferred_element_type=jnp.float32)
    z = jnp.maximum(z + b1u_ref[...], 0.0)
    o_ref[...] = (h_ref[...]
                  + jnp.dot(z, wu2_ref[...], preferred_element_type=jnp.float32)
                  + b2u_ref[...])


def _make_edge_kernel(np_rows, chunks_per_tile):
    rows_per_tile = np_rows // NS
    mesh = plsc.VectorSubcoreMesh(core_axis_name="c", subcore_axis_name="s")

    def body(p_hbm, q_hbm, src_hbm, dst_hbm, sout_hbm,
             src_v, dst_v, didx_v, prow, qrow, orow, s_sh, sem_p, sem_q):
        cid = lax.axis_index("c")
        sid = lax.axis_index("s")
        off = cid * np_rows

        zero16 = jnp.zeros((16,), jnp.float32)
        lane = lax.iota(jnp.int32, 16)
        unit = jnp.where(lane == 0, 1.0, 0.0).astype(jnp.float32)

        def zero_orow(i, carry):
            for j in range(DW // 16):
                orow[i, pl.ds(j * 16, 16)] = zero16
            return carry

        lax.fori_loop(0, CHUNK, zero_orow, 0)

        for t in range(rows_per_tile // CHUNK):
            pltpu.sync_copy(
                orow, s_sh.at[pl.ds(sid * rows_per_tile + t * CHUNK, CHUNK)])
        plsc.subcore_barrier()

        def const_row(i, carry):
            orow[i, pl.ds(HW, 16)] = unit
            return carry

        lax.fori_loop(0, CHUNK, const_row, 0)

        def chunk_body(k, carry):
            base = (sid * chunks_per_tile + k) * CHUNK
            pltpu.sync_copy(src_hbm.at[pl.ds(base, CHUNK)], src_v)
            pltpu.sync_copy(dst_hbm.at[pl.ds(base, CHUNK)], dst_v)
            for g in range(CHUNK // 16):
                s = pl.ds(g * 16, 16)
                src_v[s] = src_v[s] + off
                didx_v[s] = dst_v[s] + off
            cp = pltpu.async_copy(p_hbm.at[src_v], prow, sem_p)
            cq = pltpu.async_copy(q_hbm.at[didx_v], qrow, sem_q)
            cp.wait()
            cq.wait()

            def relu_row(i, c2):
                for j in range(HW // 16):
                    s = pl.ds(j * 16, 16)
                    orow[i, s] = jnp.maximum(prow[i, s] + qrow[i, s], 0.0)
                return c2

            lax.fori_loop(0, CHUNK, relu_row, 0)
            pltpu.sync_copy(orow, s_sh.at[dst_v], add=True)
            return carry

        lax.fori_loop(0, chunks_per_tile, chunk_body, 0)
        plsc.subcore_barrier()

        for t in range(rows_per_tile // CHUNK):
            r0 = sid * rows_per_tile + t * CHUNK
            pltpu.sync_copy(s_sh.at[pl.ds(r0, CHUNK)], orow)
            pltpu.sync_copy(orow, sout_hbm.at[cid, pl.ds(r0, CHUNK)])

    return pl.kernel(
        body,
        mesh=mesh,
        compiler_params=pltpu.CompilerParams(
            needs_layout_passes=False, use_tc_tiling_on_sc=False),
        out_type=jax.ShapeDtypeStruct((NC, np_rows, DW), jnp.float32),
        scratch_types=[
            pltpu.VMEM((CHUNK,), jnp.int32),
            pltpu.VMEM((CHUNK,), jnp.int32),
            pltpu.VMEM((CHUNK,), jnp.int32),
            pltpu.VMEM((CHUNK, HW), jnp.float32),
            pltpu.VMEM((CHUNK, HW), jnp.float32),
            pltpu.VMEM((CHUNK, DW), jnp.float32),
            pltpu.VMEM_SHARED((np_rows, DW), jnp.float32),
            pltpu.SemaphoreType.DMA,
            pltpu.SemaphoreType.DMA,
        ],
    )


def kernel(h, edge_index, W1m, b1m, W2m, b2m, W1u, b1u, W2u, b2u):
    n = h.shape[1]
    e = edge_index.shape[1]
    np_rows = -(-(n + 1) // (NS * CHUNK)) * (NS * CHUNK)
    edges_per_wave = NS * CHUNK
    e_pad = -(-e // edges_per_wave) * edges_per_wave
    chunks_per_tile = e_pad // edges_per_wave

    h0 = h[0]
    h_pad = jnp.concatenate(
        [h0, jnp.zeros((np_rows - n, H), jnp.float32)], axis=0)
    src = jnp.concatenate(
        [edge_index[0].astype(jnp.int32),
         jnp.full((e_pad - e,), n, jnp.int32)])
    dst = jnp.concatenate(
        [edge_index[1].astype(jnp.int32),
         jnp.full((e_pad - e,), n, jnp.int32)])

    at = W1m[:, :H].T
    bt = W1m[:, H:].T

    grid = (np_rows // BLK,)
    full = lambda i: (0, 0)
    rowblk = pl.BlockSpec((BLK, H), lambda i: (i, 0))
    halfblk = pl.BlockSpec((NC, BLK, HW), lambda i: (0, i, 0))
    p_nodes, q_nodes = pl.pallas_call(
        _pre_body,
        grid=grid,
        in_specs=[rowblk,
                  pl.BlockSpec((H, H), full),
                  pl.BlockSpec((H, H), full),
                  pl.BlockSpec((1, H), full)],
        out_specs=[halfblk, halfblk],
        out_shape=[jax.ShapeDtypeStruct((NC, np_rows, HW), jnp.float32)] * 2,
    )(h_pad, at, bt, b1m.reshape(1, H))

    s_part = _make_edge_kernel(np_rows, chunks_per_tile)(
        p_nodes.reshape(NC * np_rows, HW), q_nodes.reshape(NC * np_rows, HW),
        src, dst)

    w2t = W2m.T
    sblk = pl.BlockSpec((BLK, DW), lambda i: (i, 0))
    out_full = pl.pallas_call(
        _post_body,
        grid=grid,
        in_specs=[rowblk, sblk, sblk,
                  pl.BlockSpec((HW, H), full),
                  pl.BlockSpec((HW, H), full),
                  pl.BlockSpec((1, H), full),
                  pl.BlockSpec((H, H), full),
                  pl.BlockSpec((H, H), full),
                  pl.BlockSpec((1, H), full),
                  pl.BlockSpec((H, H), full),
                  pl.BlockSpec((1, H), full)],
        out_specs=rowblk,
        out_shape=jax.ShapeDtypeStruct((np_rows, H), jnp.float32),
    )(h_pad, s_part[0], s_part[1], w2t[:HW], w2t[HW:], b2m.reshape(1, H),
      W1u[:, :H].T, W1u[:, H:].T, b1u.reshape(1, H), W2u.T, b2u.reshape(1, H))

    return out_full[:n][None]


# speedup vs baseline: 3.6688x; 1.4397x over previous
import jax
import jax.numpy as jnp
from jax import lax
from jax.experimental import pallas as pl
from jax.experimental.pallas import tpu as pltpu
from jax.experimental.pallas import tpu_sc as plsc

H = 128
HW = 64
DW = 80
CHUNK = 128
SEG = 32
NC = 2
NS = 16
BLK = 1280


def _pre_body(h_ref, at_ref, bt_ref, b1_ref, p_ref, q_ref):
    x = h_ref[...]
    rp = jnp.dot(x, at_ref[...], preferred_element_type=jnp.float32)
    rq = (jnp.dot(x, bt_ref[...], preferred_element_type=jnp.float32)
          + b1_ref[...])
    p_ref[0] = rp[:, :HW]
    p_ref[1] = rp[:, HW:]
    q_ref[0] = rq[:, :HW]
    q_ref[1] = rq[:, HW:]


def _post_body(h_ref, s0_ref, s1_ref, w2a_ref, w2b_ref, b2m_ref, wua_ref,
               wub_ref, b1u_ref, wu2_ref, b2u_ref, o_ref):
    s0 = s0_ref[...]
    s1 = s1_ref[...]
    agg = jnp.dot(s0[:, :HW], w2a_ref[...], preferred_element_type=jnp.float32)
    agg = agg + jnp.dot(s1[:, :HW], w2b_ref[...],
                        preferred_element_type=jnp.float32)
    agg = agg + s0[:, HW:HW + 1] * b2m_ref[...]
    z = jnp.dot(h_ref[...], wua_ref[...], preferred_element_type=jnp.float32)
    z = z + jnp.dot(agg, wub_ref[...], preferred_element_type=jnp.float32)
    z = jnp.maximum(z + b1u_ref[...], 0.0)
    o_ref[...] = (h_ref[...]
                  + jnp.dot(z, wu2_ref[...], preferred_element_type=jnp.float32)
                  + b2u_ref[...])


def _make_edge_kernel(np_rows, chunks_per_tile):
    rows_per_tile = np_rows // NS
    ch = chunks_per_tile
    nseg = ch // SEG
    mesh = plsc.VectorSubcoreMesh(core_axis_name="c", subcore_axis_name="s")

    def body(p_hbm, q_hbm, src_hbm, dst_hbm, sout_hbm,
             sidx0, sidx1, dsct0, dsct1, didx,
             prow0, prow1, qrow0, qrow1, orow0, orow1, s_sh,
             sem_i0, sem_i1, sem_p0, sem_p1, sem_q0, sem_q1, sem_s0, sem_s1):
        cid = lax.axis_index("c")
        sid = lax.axis_index("s")
        off = cid * np_rows

        zero16 = jnp.zeros((16,), jnp.float32)
        lane = lax.iota(jnp.int32, 16)
        unit = jnp.where(lane == 0, 1.0, 0.0).astype(jnp.float32)

        def zero_orow(i, carry):
            for j in range(DW // 16):
                orow0[i, pl.ds(j * 16, 16)] = zero16
            return carry

        lax.fori_loop(0, CHUNK, zero_orow, 0)

        for t in range(rows_per_tile // CHUNK):
            pltpu.sync_copy(
                orow0, s_sh.at[pl.ds(sid * rows_per_tile + t * CHUNK, CHUNK)])
        plsc.subcore_barrier()

        def const_row(i, carry):
            orow0[i, pl.ds(HW, 16)] = unit
            orow1[i, pl.ds(HW, 16)] = unit
            return carry

        lax.fori_loop(0, CHUNK, const_row, 0)

        sidx = (sidx0, sidx1)
        dsct = (dsct0, dsct1)
        sem_i = (sem_i0, sem_i1)
        gbufs = ((prow0, qrow0, orow0, sem_p0, sem_q0, sem_s0),
                 (prow1, qrow1, orow1, sem_p1, sem_q1, sem_s1))

        def fire_idx(s):
            pb = s % 2
            base = sid * ch + s * SEG
            pltpu.async_copy(src_hbm.at[pl.ds(base, SEG)], sidx[pb], sem_i[pb])
            pltpu.async_copy(dst_hbm.at[pl.ds(base, SEG)], dsct[pb], sem_i[pb])

        def wait_idx(s):
            pb = s % 2
            pltpu.make_async_copy(
                src_hbm.at[pl.ds(0, SEG)], sidx[pb], sem_i[pb]).wait()
            pltpu.make_async_copy(
                dst_hbm.at[pl.ds(0, SEG)], dsct[pb], sem_i[pb]).wait()

        def fire_gather(pb, k, b):
            prow, qrow, _, sem_p, sem_q, _ = gbufs[b]
            pltpu.async_copy(p_hbm.at[sidx[pb].at[k]], prow, sem_p)
            pltpu.async_copy(q_hbm.at[didx.at[k]], qrow, sem_q)

        def wait_gather(pb, b):
            prow, qrow, _, sem_p, sem_q, _ = gbufs[b]
            pltpu.make_async_copy(
                p_hbm.at[sidx[pb].at[0]], prow, sem_p).wait()
            pltpu.make_async_copy(
                q_hbm.at[didx.at[0]], qrow, sem_q).wait()

        def wait_scatter(b):
            _, _, orow, _, _, sem_s = gbufs[b]
            pltpu.make_async_copy(
                orow, s_sh.at[dsct[0].at[0]], sem_s).wait()

        fire_idx(0)
        for s in range(nseg):
            pb = s % 2
            wait_idx(s)
            if s + 1 < nseg:
                fire_idx(s + 1)

            def xf(r, carry):
                for g in range(CHUNK // 16):
                    sl = pl.ds(g * 16, 16)
                    sidx[pb][r, sl] = sidx[pb][r, sl] + off
                    didx[r, sl] = dsct[pb][r, sl] + off
                return carry

            lax.fori_loop(0, SEG, xf, 0)

            fire_gather(pb, 0, 0)
            fire_gather(pb, 1, 1)

            def pair(t, carry):
                for b in (0, 1):
                    kk = 2 * t + b
                    prow, qrow, orow, sem_p, sem_q, sem_s = gbufs[b]
                    if s == 0:
                        @pl.when(t > 0)
                        def _(b=b):
                            wait_scatter(b)
                    else:
                        wait_scatter(b)
                    wait_gather(pb, b)

                    def relu_rows(iv, c2):
                        for r in range(4):
                            i = iv * 4 + r
                            for jj in range(HW // 16):
                                sl = pl.ds(jj * 16, 16)
                                orow[i, sl] = jnp.maximum(
                                    prow[i, sl] + qrow[i, sl], 0.0)
                        return c2

                    lax.fori_loop(0, CHUNK // 4, relu_rows, 0)

                    @pl.when(t < SEG // 2 - 1)
                    def _(pb=pb, kk=kk, b=b):
                        fire_gather(pb, kk + 2, b)

                    pltpu.async_copy(
                        orow, s_sh.at[dsct[pb].at[kk]], sem_s, add=True)
                return carry

            lax.fori_loop(0, SEG // 2, pair, 0)

        wait_scatter(0)
        wait_scatter(1)
        plsc.subcore_barrier()

        for t in range(rows_per_tile // CHUNK):
            r0 = sid * rows_per_tile + t * CHUNK
            pltpu.sync_copy(s_sh.at[pl.ds(r0, CHUNK)], orow0)
            pltpu.sync_copy(orow0, sout_hbm.at[cid, pl.ds(r0, CHUNK)])

    return pl.kernel(
        body,
        mesh=mesh,
        compiler_params=pltpu.CompilerParams(
            needs_layout_passes=False, use_tc_tiling_on_sc=False),
        out_type=jax.ShapeDtypeStruct((NC, np_rows, DW), jnp.float32),
        scratch_types=[
            pltpu.VMEM((SEG, CHUNK), jnp.int32),
            pltpu.VMEM((SEG, CHUNK), jnp.int32),
            pltpu.VMEM((SEG, CHUNK), jnp.int32),
            pltpu.VMEM((SEG, CHUNK), jnp.int32),
            pltpu.VMEM((SEG, CHUNK), jnp.int32),
            pltpu.VMEM((CHUNK, HW), jnp.float32),
            pltpu.VMEM((CHUNK, HW), jnp.float32),
            pltpu.VMEM((CHUNK, HW), jnp.float32),
            pltpu.VMEM((CHUNK, HW), jnp.float32),
            pltpu.VMEM((CHUNK, DW), jnp.float32),
            pltpu.VMEM((CHUNK, DW), jnp.float32),
            pltpu.VMEM_SHARED((np_rows, DW), jnp.float32),
            pltpu.SemaphoreType.DMA,
            pltpu.SemaphoreType.DMA,
            pltpu.SemaphoreType.DMA,
            pltpu.SemaphoreType.DMA,
            pltpu.SemaphoreType.DMA,
            pltpu.SemaphoreType.DMA,
            pltpu.SemaphoreType.DMA,
            pltpu.SemaphoreType.DMA,
        ],
    )


def kernel(h, edge_index, W1m, b1m, W2m, b2m, W1u, b1u, W2u, b2u):
    n = h.shape[1]
    e = edge_index.shape[1]
    np_rows = -(-(n + 1) // (NS * CHUNK)) * (NS * CHUNK)
    chunks_per_tile = -(-(-(-e // (NS * CHUNK))) // SEG) * SEG
    e_pad = chunks_per_tile * NS * CHUNK

    h0 = h[0]
    h_pad = jnp.concatenate(
        [h0, jnp.zeros((np_rows - n, H), jnp.float32)], axis=0)
    src = jnp.concatenate(
        [edge_index[0].astype(jnp.int32),
         jnp.full((e_pad - e,), n, jnp.int32)])
    dst = jnp.concatenate(
        [edge_index[1].astype(jnp.int32),
         jnp.full((e_pad - e,), n, jnp.int32)])
    nrows = NS * chunks_per_tile
    src_2d = src.reshape(nrows, CHUNK)
    dst_2d = dst.reshape(nrows, CHUNK)

    at = W1m[:, :H].T
    bt = W1m[:, H:].T

    grid = (np_rows // BLK,)
    full = lambda i: (0, 0)
    rowblk = pl.BlockSpec((BLK, H), lambda i: (i, 0))
    halfblk = pl.BlockSpec((NC, BLK, HW), lambda i: (0, i, 0))
    p_nodes, q_nodes = pl.pallas_call(
        _pre_body,
        grid=grid,
        in_specs=[rowblk,
                  pl.BlockSpec((H, H), full),
                  pl.BlockSpec((H, H), full),
                  pl.BlockSpec((1, H), full)],
        out_specs=[halfblk, halfblk],
        out_shape=[jax.ShapeDtypeStruct((NC, np_rows, HW), jnp.float32)] * 2,
    )(h_pad, at, bt, b1m.reshape(1, H))

    s_part = _make_edge_kernel(np_rows, chunks_per_tile)(
        p_nodes.reshape(NC * np_rows, HW), q_nodes.reshape(NC * np_rows, HW),
        src_2d, dst_2d)

    w2t = W2m.T
    sblk = pl.BlockSpec((BLK, DW), lambda i: (i, 0))
    out_full = pl.pallas_call(
        _post_body,
        grid=grid,
        in_specs=[rowblk, sblk, sblk,
                  pl.BlockSpec((HW, H), full),
                  pl.BlockSpec((HW, H), full),
                  pl.BlockSpec((1, H), full),
                  pl.BlockSpec((H, H), full),
                  pl.BlockSpec((H, H), full),
                  pl.BlockSpec((1, H), full),
                  pl.BlockSpec((H, H), full),
                  pl.BlockSpec((1, H), full)],
        out_specs=rowblk,
        out_shape=jax.ShapeDtypeStruct((np_rows, H), jnp.float32),
    )(h_pad, s_part[0], s_part[1], w2t[:HW], w2t[HW:], b2m.reshape(1, H),
      W1u[:, :H].T, W1u[:, H:].T, b1u.reshape(1, H), W2u.T, b2u.reshape(1, H))

    return out_full[:n][None]


# speedup vs baseline: 4.9608x; 1.3522x over previous
import jax
import jax.numpy as jnp
from jax import lax
from jax.experimental import pallas as pl
from jax.experimental.pallas import tpu as pltpu
from jax.experimental.pallas import tpu_sc as plsc

H = 128
HW = 64
CHUNK = 128
SEG = 32
NC = 2
NS = 16
BLK = 1280


def _pre_body(h_ref, at_ref, bt_ref, b1_ref, p_ref, q_ref):
    x = h_ref[...]
    rp = jnp.dot(x, at_ref[...], preferred_element_type=jnp.float32)
    rq = (jnp.dot(x, bt_ref[...], preferred_element_type=jnp.float32)
          + b1_ref[...])
    p_ref[0] = rp[:, :HW]
    p_ref[1] = rp[:, HW:]
    q_ref[0] = rq[:, :HW]
    q_ref[1] = rq[:, HW:]


def _post_body(h_ref, s0_ref, s1_ref, c_ref, w2a_ref, w2b_ref, b2m_ref,
               wua_ref, wub_ref, b1u_ref, wu2_ref, b2u_ref, o_ref):
    agg = jnp.dot(s0_ref[...], w2a_ref[...],
                  preferred_element_type=jnp.float32)
    agg = agg + jnp.dot(s1_ref[...], w2b_ref[...],
                        preferred_element_type=jnp.float32)
    c_row = jnp.sum(c_ref[...], axis=0, keepdims=True)
    agg = agg + lax.dot_general(
        c_row, b2m_ref[...], (((0,), (0,)), ((), ())),
        preferred_element_type=jnp.float32)
    z = jnp.dot(h_ref[...], wua_ref[...], preferred_element_type=jnp.float32)
    z = z + jnp.dot(agg, wub_ref[...], preferred_element_type=jnp.float32)
    z = jnp.maximum(z + b1u_ref[...], 0.0)
    o_ref[...] = (h_ref[...]
                  + jnp.dot(z, wu2_ref[...], preferred_element_type=jnp.float32)
                  + b2u_ref[...])


def _make_edge_kernel(np_rows, chunks_per_tile):
    rows_per_tile = np_rows // NS
    ch = chunks_per_tile
    nseg = ch // SEG
    mesh = plsc.VectorSubcoreMesh(core_axis_name="c", subcore_axis_name="s")

    def body(p_hbm, q_hbm, src_hbm, dst_hbm, sout_hbm, cout_hbm,
             sidx0, sidx1, dsct0, dsct1, didx, cnt_v,
             prow0, prow1, qrow0, qrow1, orow0, orow1, s_sh,
             sem_i0, sem_i1, sem_p0, sem_p1, sem_q0, sem_q1, sem_s0, sem_s1):
        cid = lax.axis_index("c")
        sid = lax.axis_index("s")
        off = cid * np_rows

        zero16 = jnp.zeros((16,), jnp.float32)
        ones16 = jnp.full((16,), 1.0, jnp.float32)
        lane = lax.iota(jnp.int32, 16)
        lane_masks = [lane == l for l in range(16)]

        def zero_orow(i, carry):
            for j in range(HW // 16):
                orow0[i, pl.ds(j * 16, 16)] = zero16
            return carry

        lax.fori_loop(0, CHUNK, zero_orow, 0)

        def zero_cnt(i, carry):
            cnt_v[pl.ds(i * 16, 16)] = zero16
            return carry

        lax.fori_loop(0, np_rows // 16, zero_cnt, 0)

        for t in range(rows_per_tile // CHUNK):
            pltpu.sync_copy(
                orow0, s_sh.at[pl.ds(sid * rows_per_tile + t * CHUNK, CHUNK)])
        plsc.subcore_barrier()

        sidx = (sidx0, sidx1)
        dsct = (dsct0, dsct1)
        sem_i = (sem_i0, sem_i1)
        gbufs = ((prow0, qrow0, orow0, sem_p0, sem_q0, sem_s0),
                 (prow1, qrow1, orow1, sem_p1, sem_q1, sem_s1))

        def fire_idx(s):
            pb = s % 2
            base = sid * ch + s * SEG
            pltpu.async_copy(src_hbm.at[pl.ds(base, SEG)], sidx[pb], sem_i[pb])
            pltpu.async_copy(dst_hbm.at[pl.ds(base, SEG)], dsct[pb], sem_i[pb])

        def wait_idx(s):
            pb = s % 2
            pltpu.make_async_copy(
                src_hbm.at[pl.ds(0, SEG)], sidx[pb], sem_i[pb]).wait()
            pltpu.make_async_copy(
                dst_hbm.at[pl.ds(0, SEG)], dsct[pb], sem_i[pb]).wait()

        def fire_gather(pb, k, b):
            prow, qrow, _, sem_p, sem_q, _ = gbufs[b]
            pltpu.async_copy(p_hbm.at[sidx[pb].at[k]], prow, sem_p)
            pltpu.async_copy(q_hbm.at[didx.at[k]], qrow, sem_q)

        def wait_gather(pb, b):
            prow, qrow, _, sem_p, sem_q, _ = gbufs[b]
            pltpu.make_async_copy(
                p_hbm.at[sidx[pb].at[0]], prow, sem_p).wait()
            pltpu.make_async_copy(
                q_hbm.at[didx.at[0]], qrow, sem_q).wait()

        def wait_scatter(b):
            _, _, orow, _, _, sem_s = gbufs[b]
            pltpu.make_async_copy(
                orow, s_sh.at[dsct[0].at[0]], sem_s).wait()

        fire_idx(0)
        for s in range(nseg):
            pb = s % 2
            wait_idx(s)
            if s + 1 < nseg:
                fire_idx(s + 1)

            def xf(r, carry):
                for g in range(CHUNK // 16):
                    sl = pl.ds(g * 16, 16)
                    sidx[pb][r, sl] = sidx[pb][r, sl] + off
                    didx[r, sl] = dsct[pb][r, sl] + off
                return carry

            lax.fori_loop(0, SEG, xf, 0)

            fire_gather(pb, 0, 0)
            fire_gather(pb, 1, 1)

            def pair(t, carry):
                for b in (0, 1):
                    kk = 2 * t + b
                    prow, qrow, orow, sem_p, sem_q, sem_s = gbufs[b]
                    if s == 0:
                        @pl.when(t > 0)
                        def _(b=b):
                            wait_scatter(b)
                    else:
                        wait_scatter(b)
                    wait_gather(pb, b)

                    for g in range(CHUNK // 16):
                        dvec = dsct[pb][kk, pl.ds(g * 16, 16)]
                        for msk in lane_masks:
                            plsc.addupdate_scatter(
                                cnt_v, [dvec], ones16, mask=msk)

                    def relu_rows(iv, c2):
                        for r in range(4):
                            i = iv * 4 + r
                            for jj in range(HW // 16):
                                sl = pl.ds(jj * 16, 16)
                                orow[i, sl] = jnp.maximum(
                                    prow[i, sl] + qrow[i, sl], 0.0)
                        return c2

                    lax.fori_loop(0, CHUNK // 4, relu_rows, 0)

                    @pl.when(t < SEG // 2 - 1)
                    def _(pb=pb, kk=kk, b=b):
                        fire_gather(pb, kk + 2, b)

                    pltpu.async_copy(
                        orow, s_sh.at[dsct[pb].at[kk]], sem_s, add=True)
                return carry

            lax.fori_loop(0, SEG // 2, pair, 0)

        wait_scatter(0)
        wait_scatter(1)
        pltpu.sync_copy(cnt_v, cout_hbm.at[cid * NS + sid])
        plsc.subcore_barrier()

        for t in range(rows_per_tile // CHUNK):
            r0 = sid * rows_per_tile + t * CHUNK
            pltpu.sync_copy(s_sh.at[pl.ds(r0, CHUNK)], orow0)
            pltpu.sync_copy(orow0, sout_hbm.at[cid, pl.ds(r0, CHUNK)])

    return pl.kernel(
        body,
        mesh=mesh,
        compiler_params=pltpu.CompilerParams(
            needs_layout_passes=False, use_tc_tiling_on_sc=False),
        out_type=[
            jax.ShapeDtypeStruct((NC, np_rows, HW), jnp.float32),
            jax.ShapeDtypeStruct((NC * NS, np_rows), jnp.float32),
        ],
        scratch_types=[
            pltpu.VMEM((SEG, CHUNK), jnp.int32),
            pltpu.VMEM((SEG, CHUNK), jnp.int32),
            pltpu.VMEM((SEG, CHUNK), jnp.int32),
            pltpu.VMEM((SEG, CHUNK), jnp.int32),
            pltpu.VMEM((SEG, CHUNK), jnp.int32),
            pltpu.VMEM((np_rows,), jnp.float32),
            pltpu.VMEM((CHUNK, HW), jnp.float32),
            pltpu.VMEM((CHUNK, HW), jnp.float32),
            pltpu.VMEM((CHUNK, HW), jnp.float32),
            pltpu.VMEM((CHUNK, HW), jnp.float32),
            pltpu.VMEM((CHUNK, HW), jnp.float32),
            pltpu.VMEM((CHUNK, HW), jnp.float32),
            pltpu.VMEM_SHARED((np_rows, HW), jnp.float32),
            pltpu.SemaphoreType.DMA,
            pltpu.SemaphoreType.DMA,
            pltpu.SemaphoreType.DMA,
            pltpu.SemaphoreType.DMA,
            pltpu.SemaphoreType.DMA,
            pltpu.SemaphoreType.DMA,
            pltpu.SemaphoreType.DMA,
            pltpu.SemaphoreType.DMA,
        ],
    )


def kernel(h, edge_index, W1m, b1m, W2m, b2m, W1u, b1u, W2u, b2u):
    n = h.shape[1]
    e = edge_index.shape[1]
    np_rows = -(-(n + 1) // (NS * CHUNK)) * (NS * CHUNK)
    chunks_per_tile = -(-(-(-e // (NS * CHUNK))) // SEG) * SEG
    e_pad = chunks_per_tile * NS * CHUNK

    h0 = h[0]
    h_pad = jnp.concatenate(
        [h0, jnp.zeros((np_rows - n, H), jnp.float32)], axis=0)
    src = jnp.concatenate(
        [edge_index[0].astype(jnp.int32),
         jnp.full((e_pad - e,), n, jnp.int32)])
    dst = jnp.concatenate(
        [edge_index[1].astype(jnp.int32),
         jnp.full((e_pad - e,), n, jnp.int32)])
    nrows = NS * chunks_per_tile
    src_2d = src.reshape(nrows, CHUNK)
    dst_2d = dst.reshape(nrows, CHUNK)

    at = W1m[:, :H].T
    bt = W1m[:, H:].T

    grid = (np_rows // BLK,)
    full = lambda i: (0, 0)
    rowblk = pl.BlockSpec((BLK, H), lambda i: (i, 0))
    halfblk = pl.BlockSpec((NC, BLK, HW), lambda i: (0, i, 0))
    p_nodes, q_nodes = pl.pallas_call(
        _pre_body,
        grid=grid,
        in_specs=[rowblk,
                  pl.BlockSpec((H, H), full),
                  pl.BlockSpec((H, H), full),
                  pl.BlockSpec((1, H), full)],
        out_specs=[halfblk, halfblk],
        out_shape=[jax.ShapeDtypeStruct((NC, np_rows, HW), jnp.float32)] * 2,
    )(h_pad, at, bt, b1m.reshape(1, H))

    s_part, c_part = _make_edge_kernel(np_rows, chunks_per_tile)(
        p_nodes.reshape(NC * np_rows, HW), q_nodes.reshape(NC * np_rows, HW),
        src_2d, dst_2d)

    w2t = W2m.T
    sblk = pl.BlockSpec((BLK, HW), lambda i: (i, 0))
    out_full = pl.pallas_call(
        _post_body,
        grid=grid,
        in_specs=[rowblk, sblk, sblk,
                  pl.BlockSpec((NS, BLK), lambda i: (0, i)),
                  pl.BlockSpec((HW, H), full),
                  pl.BlockSpec((HW, H), full),
                  pl.BlockSpec((1, H), full),
                  pl.BlockSpec((H, H), full),
                  pl.BlockSpec((H, H), full),
                  pl.BlockSpec((1, H), full),
                  pl.BlockSpec((H, H), full),
                  pl.BlockSpec((1, H), full)],
        out_specs=rowblk,
        out_shape=jax.ShapeDtypeStruct((np_rows, H), jnp.float32),
    )(h_pad, s_part[0], s_part[1], c_part[:NS], w2t[:HW], w2t[HW:],
      b2m.reshape(1, H), W1u[:, :H].T, W1u[:, H:].T, b1u.reshape(1, H),
      W2u.T, b2u.reshape(1, H))

    return out_full[:n][None]
